# baseline (speedup 1.0000x reference)
import jax
import jax.numpy as jnp
from jax import lax
from jax.experimental import pallas as pl
from jax.experimental.pallas import tpu as pltpu
from jax.experimental.pallas import tpu_sc as plsc

N_NODES = 10000
N_EDGES = 320000
D = 128
DH = D // 2

NC = 2
NS = 16

EDGES_PER_T = N_EDGES // NS
CHUNK = 128
SEG = 4096
NSEG = 5
SEG_CHUNKS = (32, 32, 32, 32, 28)
SEG_SIZES = (4096, 4096, 4096, 4096, 3616)
TAIL = 32
ROWS_PER_SC = N_NODES // NS
ACC_ROWS = 10112
ROWS_PER_TILE = ACC_ROWS // NS
DEG_W = 8
NB = 2


def _sc_body(edge_hbm, feat2_hbm, zacc_hbm, zdeg_hbm, ones_hbm,
             acc_out, deg_out,
             srcA, dstA, srcB, dstB, b0, b1, ones_v, table_sh, acc_sh, deg_sh,
             g0, g1, s0, s1, d0, d1, ia, ib):
    c = lax.axis_index("c")
    s = lax.axis_index("s")
    bufs = (b0, b1)
    gsem = (g0, g1)
    ssem = (s0, s1)
    dsem = (d0, d1)
    isem = (ia, ib)
    seg_src = (srcA, srcB)
    seg_dst = (dstA, dstB)

    pltpu.sync_copy(feat2_hbm.at[c].at[pl.ds(s * ROWS_PER_SC, ROWS_PER_SC)],
                    table_sh.at[pl.ds(s * ROWS_PER_SC, ROWS_PER_SC)])
    o0 = s * ROWS_PER_TILE
    pltpu.sync_copy(zacc_hbm, acc_sh.at[pl.ds(o0, ROWS_PER_TILE)])
    pltpu.sync_copy(zdeg_hbm, deg_sh.at[pl.ds(o0, ROWS_PER_TILE)])
    pltpu.sync_copy(ones_hbm, ones_v)
    e0 = s * EDGES_PER_T
    pltpu.sync_copy(edge_hbm.at[0, pl.ds(e0, SEG)], srcA)
    pltpu.sync_copy(edge_hbm.at[1, pl.ds(e0, SEG)], dstA)
    plsc.subcore_barrier()

    def make_ops(src_v, dst_v):
        def gstart(j, b):
            pltpu.async_copy(table_sh.at[src_v.at[pl.ds(j * CHUNK, CHUNK)]],
                             bufs[b], gsem[b])

        def gwait(j, b):
            pltpu.make_async_copy(
                table_sh.at[src_v.at[pl.ds(j * CHUNK, CHUNK)]],
                bufs[b], gsem[b]).wait()

        def sstart(j, b, first_deg=False):
            pltpu.async_copy(bufs[b],
                             acc_sh.at[dst_v.at[pl.ds(j * CHUNK, CHUNK)]],
                             ssem[b], add=True)
            @pl.when(c == b)
            def _():
                if not first_deg:
                    pltpu.make_async_copy(
                        ones_v, deg_sh.at[dst_v.at[pl.ds(j * CHUNK, CHUNK)]],
                        dsem[b]).wait()
                pltpu.async_copy(ones_v,
                                 deg_sh.at[dst_v.at[pl.ds(j * CHUNK, CHUNK)]],
                                 dsem[b], add=True)

        def swait(j, b):
            pltpu.make_async_copy(bufs[b],
                                  acc_sh.at[dst_v.at[pl.ds(j * CHUNK, CHUNK)]],
                                  ssem[b]).wait()

        return gstart, gwait, sstart, swait

    for seg in range(NSEG):
        src_v = seg_src[seg % 2]
        dst_v = seg_dst[seg % 2]
        gstart, gwait, sstart, swait = make_ops(src_v, dst_v)
        if seg > 0:
            off = e0 + seg * SEG
            sz = SEG_SIZES[seg]
            pltpu.make_async_copy(edge_hbm.at[0, pl.ds(off, sz)],
                                  src_v.at[pl.ds(0, sz)], isem[seg % 2]).wait()
            pltpu.make_async_copy(edge_hbm.at[1, pl.ds(off, sz)],
                                  dst_v.at[pl.ds(0, sz)], isem[seg % 2]).wait()
        if seg < NSEG - 1:
            noff = e0 + (seg + 1) * SEG
            nsize = SEG_SIZES[seg + 1]
            pltpu.async_copy(edge_hbm.at[0, pl.ds(noff, nsize)],
                             seg_src[(seg + 1) % 2].at[pl.ds(0, nsize)],
                             isem[(seg + 1) % 2])
            pltpu.async_copy(edge_hbm.at[1, pl.ds(noff, nsize)],
                             seg_dst[(seg + 1) % 2].at[pl.ds(0, nsize)],
                             isem[(seg + 1) % 2])

        nch = SEG_CHUNKS[seg]
        gstart(0, 0)
        gstart(1, 1)
        gwait(0, 0)
        sstart(0, 0, first_deg=(seg == 0))
        gwait(1, 1)
        sstart(1, 1, first_deg=(seg == 0))
        swait(0, 0)
        gstart(2, 0)

        def pair(j2, carry):
            j = 2 * j2
            gwait(j, 0)
            sstart(j, 0)
            swait(j - 1, 1)
            gstart(j + 1, 1)
            gwait(j + 1, 1)
            sstart(j + 1, 1)
            swait(j, 0)
            gstart(j + 2, 0)
            return carry

        lax.fori_loop(1, nch // 2 - 1, pair, 0)
        j = nch - 2
        gwait(j, 0)
        sstart(j, 0)
        swait(j - 1, 1)
        gstart(j + 1, 1)
        gwait(j + 1, 1)
        sstart(j + 1, 1)
        swait(j, 0)
        swait(j + 1, 1)

    src_v = seg_src[(NSEG - 1) % 2]
    dst_v = seg_dst[(NSEG - 1) % 2]
    t0 = SEG_CHUNKS[NSEG - 1] * CHUNK
    pltpu.async_copy(table_sh.at[src_v.at[pl.ds(t0, TAIL)]],
                     b0.at[pl.ds(0, TAIL)], g0).wait()
    pltpu.sync_copy(b0.at[pl.ds(0, TAIL)],
                    acc_sh.at[dst_v.at[pl.ds(t0, TAIL)]], add=True)

    @pl.when(c == 0)
    def _():
        pltpu.sync_copy(ones_v.at[pl.ds(0, TAIL)],
                        deg_sh.at[dst_v.at[pl.ds(t0, TAIL)]], add=True)

    for b in range(NB):
        @pl.when(c == b)
        def _(b=b):
            pltpu.make_async_copy(
                ones_v, deg_sh.at[seg_dst[0].at[pl.ds(0, CHUNK)]],
                dsem[b]).wait()

    plsc.subcore_barrier()

    pltpu.sync_copy(acc_sh.at[pl.ds(o0, ROWS_PER_TILE)],
                    acc_out.at[c, pl.ds(o0, ROWS_PER_TILE)])
    pltpu.sync_copy(deg_sh.at[pl.ds(o0, ROWS_PER_TILE)],
                    deg_out.at[c, pl.ds(o0, ROWS_PER_TILE)])


_sc_aggregate = pl.kernel(
    _sc_body,
    out_type=(
        jax.ShapeDtypeStruct((NC, ACC_ROWS, DH), jnp.float32),
        jax.ShapeDtypeStruct((NC, ACC_ROWS, DEG_W), jnp.float32),
    ),
    mesh=plsc.VectorSubcoreMesh(core_axis_name="c", subcore_axis_name="s"),
    compiler_params=pltpu.CompilerParams(use_tc_tiling_on_sc=False),
    scratch_types=[
        pltpu.VMEM((SEG,), jnp.int32),
        pltpu.VMEM((SEG,), jnp.int32),
        pltpu.VMEM((SEG,), jnp.int32),
        pltpu.VMEM((SEG,), jnp.int32),
        pltpu.VMEM((CHUNK, DH), jnp.float32),
        pltpu.VMEM((CHUNK, DH), jnp.float32),
        pltpu.VMEM((CHUNK, DEG_W), jnp.float32),
        pltpu.VMEM_SHARED((N_NODES, DH), jnp.float32),
        pltpu.VMEM_SHARED((ACC_ROWS, DH), jnp.float32),
        pltpu.VMEM_SHARED((ACC_ROWS, DEG_W), jnp.float32),
        pltpu.SemaphoreType.DMA,
        pltpu.SemaphoreType.DMA,
        pltpu.SemaphoreType.DMA,
        pltpu.SemaphoreType.DMA,
        pltpu.SemaphoreType.DMA,
        pltpu.SemaphoreType.DMA,
        pltpu.SemaphoreType.DMA,
        pltpu.SemaphoreType.DMA,
    ],
)


def _tc_body(feat_ref, acc_ref, deg_ref, wnl_ref, wnr_ref, ws_ref, b_ref,
             o_ref):
    deg = deg_ref[0, :, :1] + deg_ref[1, :, :1]
    inv = 1.0 / jnp.maximum(deg, 1.0)
    h0 = acc_ref[0] * inv
    h1 = acc_ref[1] * inv
    x = feat_ref[...]
    dn = (((1,), (1,)), ((), ()))
    o_ref[...] = (
        lax.dot_general(x, ws_ref[...], dn, preferred_element_type=jnp.float32)
        + lax.dot_general(h0, wnl_ref[...], dn, preferred_element_type=jnp.float32)
        + lax.dot_general(h1, wnr_ref[...], dn, preferred_element_type=jnp.float32)
        + b_ref[...]
    )


ROW_BLK = 2000

_tc_combine = pl.pallas_call(
    _tc_body,
    grid=(N_NODES // ROW_BLK,),
    in_specs=[
        pl.BlockSpec((ROW_BLK, D), lambda i: (i, 0)),
        pl.BlockSpec((NC, ROW_BLK, DH), lambda i: (0, i, 0)),
        pl.BlockSpec((NC, ROW_BLK, DEG_W), lambda i: (0, i, 0)),
        pl.BlockSpec((D, DH), lambda i: (0, 0)),
        pl.BlockSpec((D, DH), lambda i: (0, 0)),
        pl.BlockSpec((D, D), lambda i: (0, 0)),
        pl.BlockSpec((1, D), lambda i: (0, 0)),
    ],
    out_specs=pl.BlockSpec((ROW_BLK, D), lambda i: (i, 0)),
    out_shape=jax.ShapeDtypeStruct((N_NODES, D), jnp.float32),
)


@jax.jit
def kernel(feat, edge_index, W_neigh, W_self, b_self):
    feat2 = jnp.stack([feat[:, :DH], feat[:, DH:]])
    zacc = jnp.zeros((ROWS_PER_TILE, DH), jnp.float32)
    zdeg = jnp.zeros((ROWS_PER_TILE, DEG_W), jnp.float32)
    ones = jnp.ones((CHUNK, DEG_W), jnp.float32)

    acc, deg = _sc_aggregate(edge_index, feat2, zacc, zdeg, ones)

    return _tc_combine(feat, acc, deg,
                       W_neigh[:, :DH], W_neigh[:, DH:],
                       W_self, b_self.reshape(1, D))


# speedup vs baseline: 1.5434x; 1.5434x over previous
import jax
import jax.numpy as jnp
from jax import lax
from jax.experimental import pallas as pl
from jax.experimental.pallas import tpu as pltpu
from jax.experimental.pallas import tpu_sc as plsc

N_NODES = 10000
N_EDGES = 320000
D = 128
DH = D // 2

NC = 2
NS = 16

EDGES_PER_T = N_EDGES // NS
CHUNK = 128
CHUNKS_PER_T = 156
TAIL = EDGES_PER_T - CHUNKS_PER_T * CHUNK
ACC_ROWS = 10112
ROWS_PER_TILE = ACC_ROWS // NS
DEG_W = 8
NB = 4


def _sc_body(edge_hbm, table_hbm, zacc_hbm, zdeg_hbm, ones_hbm,
             acc_out, deg_out,
             src_v, dst_v, b0, b1, b2, b3, ones_v, acc_sh, deg_sh,
             g0, g1, g2, g3, s0, s1, s2, s3, d0, d1, d2, d3):
    c = lax.axis_index("c")
    s = lax.axis_index("s")
    bufs = (b0, b1, b2, b3)
    gsem = (g0, g1, g2, g3)
    ssem = (s0, s1, s2, s3)
    dsem = (d0, d1, d2, d3)

    o0 = s * ROWS_PER_TILE
    pltpu.sync_copy(zacc_hbm, acc_sh.at[pl.ds(o0, ROWS_PER_TILE)])
    pltpu.sync_copy(zdeg_hbm, deg_sh.at[pl.ds(o0, ROWS_PER_TILE)])
    pltpu.sync_copy(ones_hbm, ones_v)
    e0 = s * EDGES_PER_T
    pltpu.sync_copy(edge_hbm.at[0, pl.ds(e0, EDGES_PER_T)], src_v)
    pltpu.sync_copy(edge_hbm.at[1, pl.ds(e0, EDGES_PER_T)], dst_v)

    def ixform(i, carry):
        v = src_v[pl.ds(i * 16, 16)]
        src_v[pl.ds(i * 16, 16)] = v * 2 + c
        return carry

    lax.fori_loop(0, EDGES_PER_T // 16, ixform, 0)
    plsc.subcore_barrier()

    def gstart(j, b):
        pltpu.async_copy(table_hbm.at[src_v.at[pl.ds(j * CHUNK, CHUNK)]],
                         bufs[b], gsem[b])

    def gwait(j, b):
        pltpu.make_async_copy(table_hbm.at[src_v.at[pl.ds(j * CHUNK, CHUNK)]],
                              bufs[b], gsem[b]).wait()

    def sstart(j, b, dwait_j=None):
        pltpu.async_copy(bufs[b], acc_sh.at[dst_v.at[pl.ds(j * CHUNK, CHUNK)]],
                         ssem[b], add=True)
        @pl.when(c == (b % 2))
        def _():
            if dwait_j is not None:
                pltpu.make_async_copy(
                    ones_v, deg_sh.at[dst_v.at[pl.ds(dwait_j * CHUNK, CHUNK)]],
                    dsem[b]).wait()
            pltpu.async_copy(ones_v,
                             deg_sh.at[dst_v.at[pl.ds(j * CHUNK, CHUNK)]],
                             dsem[b], add=True)

    def swait(j, b):
        pltpu.make_async_copy(bufs[b],
                              acc_sh.at[dst_v.at[pl.ds(j * CHUNK, CHUNK)]],
                              ssem[b]).wait()

    for j in range(NB):
        gstart(j, j)

    gwait(0, 0)
    sstart(0, 0)
    for b in range(1, NB):
        gwait(b, b)
        sstart(b, b)
        swait(b - 1, b - 1)
        gstart(b + NB - 1, b - 1)

    def ring_shift(j4, carry):
        for b in range(NB):
            j = NB * j4 + b
            prev = (b - 1) % NB
            gwait(j, b)
            sstart(j, b, dwait_j=j - NB)
            swait(j - 1, prev)
            gstart(j + NB - 1, prev)
        return carry

    lax.fori_loop(1, CHUNKS_PER_T // NB - 1, ring_shift, 0)
    last = CHUNKS_PER_T - NB
    for b in range(NB):
        j = last + b
        prev = (b - 1) % NB
        gwait(j, b)
        sstart(j, b, dwait_j=j - NB)
        swait(j - 1, prev)
        if b == 0:
            gstart(CHUNKS_PER_T - 1, NB - 1)
    swait(CHUNKS_PER_T - 1, NB - 1)
    for b in range(NB):
        @pl.when(c == (b % 2))
        def _(b=b):
            pltpu.make_async_copy(
                ones_v, deg_sh.at[dst_v.at[pl.ds((last + b) * CHUNK, CHUNK)]],
                dsem[b]).wait()

    t0 = CHUNKS_PER_T * CHUNK
    pltpu.async_copy(table_hbm.at[src_v.at[pl.ds(t0, TAIL)]],
                     b0.at[pl.ds(0, TAIL)], g0).wait()
    pltpu.sync_copy(b0.at[pl.ds(0, TAIL)],
                    acc_sh.at[dst_v.at[pl.ds(t0, TAIL)]], add=True)

    @pl.when(c == 0)
    def _():
        pltpu.sync_copy(ones_v.at[pl.ds(0, TAIL)],
                        deg_sh.at[dst_v.at[pl.ds(t0, TAIL)]], add=True)

    plsc.subcore_barrier()

    pltpu.sync_copy(acc_sh.at[pl.ds(o0, ROWS_PER_TILE)],
                    acc_out.at[c, pl.ds(o0, ROWS_PER_TILE)])
    pltpu.sync_copy(deg_sh.at[pl.ds(o0, ROWS_PER_TILE)],
                    deg_out.at[c, pl.ds(o0, ROWS_PER_TILE)])


_sc_aggregate = pl.kernel(
    _sc_body,
    out_type=(
        jax.ShapeDtypeStruct((NC, ACC_ROWS, DH), jnp.float32),
        jax.ShapeDtypeStruct((NC, ACC_ROWS, DEG_W), jnp.float32),
    ),
    mesh=plsc.VectorSubcoreMesh(core_axis_name="c", subcore_axis_name="s"),
    compiler_params=pltpu.CompilerParams(use_tc_tiling_on_sc=False),
    scratch_types=[
        pltpu.VMEM((EDGES_PER_T,), jnp.int32),
        pltpu.VMEM((EDGES_PER_T,), jnp.int32),
        pltpu.VMEM((CHUNK, DH), jnp.float32),
        pltpu.VMEM((CHUNK, DH), jnp.float32),
        pltpu.VMEM((CHUNK, DH), jnp.float32),
        pltpu.VMEM((CHUNK, DH), jnp.float32),
        pltpu.VMEM((CHUNK, DEG_W), jnp.float32),
        pltpu.VMEM_SHARED((ACC_ROWS, DH), jnp.float32),
        pltpu.VMEM_SHARED((ACC_ROWS, DEG_W), jnp.float32),
        pltpu.SemaphoreType.DMA,
        pltpu.SemaphoreType.DMA,
        pltpu.SemaphoreType.DMA,
        pltpu.SemaphoreType.DMA,
        pltpu.SemaphoreType.DMA,
        pltpu.SemaphoreType.DMA,
        pltpu.SemaphoreType.DMA,
        pltpu.SemaphoreType.DMA,
        pltpu.SemaphoreType.DMA,
        pltpu.SemaphoreType.DMA,
        pltpu.SemaphoreType.DMA,
        pltpu.SemaphoreType.DMA,
    ],
)


def _tc_self_body(feat_ref, ws_ref, b_ref, o_ref):
    dn = (((1,), (1,)), ((), ()))
    o_ref[...] = lax.dot_general(
        feat_ref[...], ws_ref[...], dn,
        preferred_element_type=jnp.float32) + b_ref[...]


def _tc_neigh_body(hself_ref, acc_ref, deg_ref, wnl_ref, wnr_ref, o_ref):
    deg = deg_ref[0, :, :1] + deg_ref[1, :, :1]
    inv = 1.0 / jnp.maximum(deg, 1.0)
    h0 = acc_ref[0] * inv
    h1 = acc_ref[1] * inv
    dn = (((1,), (1,)), ((), ()))
    o_ref[...] = (
        hself_ref[...]
        + lax.dot_general(h0, wnl_ref[...], dn, preferred_element_type=jnp.float32)
        + lax.dot_general(h1, wnr_ref[...], dn, preferred_element_type=jnp.float32)
    )


ROW_BLK = 2000

_tc_self = pl.pallas_call(
    _tc_self_body,
    grid=(N_NODES // ROW_BLK,),
    in_specs=[
        pl.BlockSpec((ROW_BLK, D), lambda i: (i, 0)),
        pl.BlockSpec((D, D), lambda i: (0, 0)),
        pl.BlockSpec((1, D), lambda i: (0, 0)),
    ],
    out_specs=pl.BlockSpec((ROW_BLK, D), lambda i: (i, 0)),
    out_shape=jax.ShapeDtypeStruct((N_NODES, D), jnp.float32),
)

_tc_neigh = pl.pallas_call(
    _tc_neigh_body,
    grid=(N_NODES // ROW_BLK,),
    in_specs=[
        pl.BlockSpec((ROW_BLK, D), lambda i: (i, 0)),
        pl.BlockSpec((NC, ROW_BLK, DH), lambda i: (0, i, 0)),
        pl.BlockSpec((NC, ROW_BLK, DEG_W), lambda i: (0, i, 0)),
        pl.BlockSpec((D, DH), lambda i: (0, 0)),
        pl.BlockSpec((D, DH), lambda i: (0, 0)),
    ],
    out_specs=pl.BlockSpec((ROW_BLK, D), lambda i: (i, 0)),
    out_shape=jax.ShapeDtypeStruct((N_NODES, D), jnp.float32),
)


@jax.jit
def kernel(feat, edge_index, W_neigh, W_self, b_self):
    table = feat.reshape(N_NODES * 2, DH)
    zacc = jnp.zeros((ROWS_PER_TILE, DH), jnp.float32)
    zdeg = jnp.zeros((ROWS_PER_TILE, DEG_W), jnp.float32)
    ones = jnp.ones((CHUNK, DEG_W), jnp.float32)

    acc, deg = _sc_aggregate(edge_index, table, zacc, zdeg, ones)
    h_self = _tc_self(feat, W_self, b_self.reshape(1, D))

    return _tc_neigh(h_self, acc, deg, W_neigh[:, :DH], W_neigh[:, DH:])


# speedup vs baseline: 1.5945x; 1.0331x over previous
import jax
import jax.numpy as jnp
from jax import lax
from jax.experimental import pallas as pl
from jax.experimental.pallas import tpu as pltpu
from jax.experimental.pallas import tpu_sc as plsc

N_NODES = 10000
N_EDGES = 320000
D = 128
DH = D // 2

NC = 2
NS = 16

EDGES_PER_T = N_EDGES // NS
CHUNK = 128
CHUNKS_PER_T = 156
TAIL = EDGES_PER_T - CHUNKS_PER_T * CHUNK
ACC_ROWS = 10112
ROWS_PER_TILE = ACC_ROWS // NS
DEG_W = 8
NB = 4


def _sc_body(edge_hbm, table_hbm, zacc_hbm, zdeg_hbm, ones_hbm,
             acc_out, deg_out,
             src_v, dst_v, b0, b1, b2, b3, ones_v, acc_sh, deg_sh,
             g0, g1, g2, g3, s0, s1, s2, s3, d0, d1, d2, d3):
    c = lax.axis_index("c")
    s = lax.axis_index("s")
    bufs = (b0, b1, b2, b3)
    gsem = (g0, g1, g2, g3)
    ssem = (s0, s1, s2, s3)
    dsem = (d0, d1, d2, d3)

    o0 = s * ROWS_PER_TILE
    pltpu.sync_copy(zacc_hbm, acc_sh.at[pl.ds(o0, ROWS_PER_TILE)])
    pltpu.sync_copy(zdeg_hbm, deg_sh.at[pl.ds(o0, ROWS_PER_TILE)])
    pltpu.sync_copy(ones_hbm, ones_v)
    e0 = s * EDGES_PER_T
    pltpu.sync_copy(edge_hbm.at[0, pl.ds(e0, EDGES_PER_T)], src_v)
    pltpu.sync_copy(edge_hbm.at[1, pl.ds(e0, EDGES_PER_T)], dst_v)

    def ixform(i, carry):
        v = src_v[pl.ds(i * 16, 16)]
        src_v[pl.ds(i * 16, 16)] = v * 2 + c
        return carry

    lax.fori_loop(0, EDGES_PER_T // 16, ixform, 0)
    plsc.subcore_barrier()

    def gstart(j, b):
        pltpu.async_copy(table_hbm.at[src_v.at[pl.ds(j * CHUNK, CHUNK)]],
                         bufs[b], gsem[b])

    def gwait(j, b):
        pltpu.make_async_copy(table_hbm.at[src_v.at[pl.ds(j * CHUNK, CHUNK)]],
                              bufs[b], gsem[b]).wait()

    def sstart(j, b, dwait_j=None):
        pltpu.async_copy(bufs[b], acc_sh.at[dst_v.at[pl.ds(j * CHUNK, CHUNK)]],
                         ssem[b], add=True)
        @pl.when(c == (b % 2))
        def _():
            if dwait_j is not None:
                pltpu.make_async_copy(
                    ones_v, deg_sh.at[dst_v.at[pl.ds(dwait_j * CHUNK, CHUNK)]],
                    dsem[b]).wait()
            pltpu.async_copy(ones_v,
                             deg_sh.at[dst_v.at[pl.ds(j * CHUNK, CHUNK)]],
                             dsem[b], add=True)

    def swait(j, b):
        pltpu.make_async_copy(bufs[b],
                              acc_sh.at[dst_v.at[pl.ds(j * CHUNK, CHUNK)]],
                              ssem[b]).wait()

    for j in range(NB):
        gstart(j, j)

    gwait(0, 0)
    sstart(0, 0)
    for b in range(1, NB):
        gwait(b, b)
        sstart(b, b)
        swait(b - 1, b - 1)
        gstart(b + NB - 1, b - 1)

    def ring_shift(j4, carry):
        for b in range(NB):
            j = NB * j4 + b
            prev = (b - 1) % NB
            gwait(j, b)
            sstart(j, b, dwait_j=j - NB)
            swait(j - 1, prev)
            gstart(j + NB - 1, prev)
        return carry

    lax.fori_loop(1, CHUNKS_PER_T // NB - 1, ring_shift, 0)
    last = CHUNKS_PER_T - NB
    for b in range(NB):
        j = last + b
        prev = (b - 1) % NB
        gwait(j, b)
        sstart(j, b, dwait_j=j - NB)
        swait(j - 1, prev)
        if b == 0:
            gstart(CHUNKS_PER_T - 1, NB - 1)
    swait(CHUNKS_PER_T - 1, NB - 1)
    for b in range(NB):
        @pl.when(c == (b % 2))
        def _(b=b):
            pltpu.make_async_copy(
                ones_v, deg_sh.at[dst_v.at[pl.ds((last + b) * CHUNK, CHUNK)]],
                dsem[b]).wait()

    t0 = CHUNKS_PER_T * CHUNK
    pltpu.async_copy(table_hbm.at[src_v.at[pl.ds(t0, TAIL)]],
                     b0.at[pl.ds(0, TAIL)], g0).wait()
    pltpu.sync_copy(b0.at[pl.ds(0, TAIL)],
                    acc_sh.at[dst_v.at[pl.ds(t0, TAIL)]], add=True)

    @pl.when(c == 0)
    def _():
        pltpu.sync_copy(ones_v.at[pl.ds(0, TAIL)],
                        deg_sh.at[dst_v.at[pl.ds(t0, TAIL)]], add=True)

    plsc.subcore_barrier()

    pltpu.sync_copy(acc_sh.at[pl.ds(o0, ROWS_PER_TILE)],
                    acc_out.at[c, pl.ds(o0, ROWS_PER_TILE)])
    pltpu.sync_copy(deg_sh.at[pl.ds(o0, ROWS_PER_TILE)],
                    deg_out.at[c, pl.ds(o0, ROWS_PER_TILE)])


_sc_aggregate = pl.kernel(
    _sc_body,
    out_type=(
        jax.ShapeDtypeStruct((NC, ACC_ROWS, DH), jnp.float32),
        jax.ShapeDtypeStruct((NC, ACC_ROWS, DEG_W), jnp.float32),
    ),
    mesh=plsc.VectorSubcoreMesh(core_axis_name="c", subcore_axis_name="s"),
    compiler_params=pltpu.CompilerParams(use_tc_tiling_on_sc=False),
    scratch_types=[
        pltpu.VMEM((EDGES_PER_T,), jnp.int32),
        pltpu.VMEM((EDGES_PER_T,), jnp.int32),
        pltpu.VMEM((CHUNK, DH), jnp.float32),
        pltpu.VMEM((CHUNK, DH), jnp.float32),
        pltpu.VMEM((CHUNK, DH), jnp.float32),
        pltpu.VMEM((CHUNK, DH), jnp.float32),
        pltpu.VMEM((CHUNK, DEG_W), jnp.float32),
        pltpu.VMEM_SHARED((ACC_ROWS, DH), jnp.float32),
        pltpu.VMEM_SHARED((ACC_ROWS, DEG_W), jnp.float32),
        pltpu.SemaphoreType.DMA,
        pltpu.SemaphoreType.DMA,
        pltpu.SemaphoreType.DMA,
        pltpu.SemaphoreType.DMA,
        pltpu.SemaphoreType.DMA,
        pltpu.SemaphoreType.DMA,
        pltpu.SemaphoreType.DMA,
        pltpu.SemaphoreType.DMA,
        pltpu.SemaphoreType.DMA,
        pltpu.SemaphoreType.DMA,
        pltpu.SemaphoreType.DMA,
        pltpu.SemaphoreType.DMA,
    ],
)


def _tc_body(feat_ref, acc_ref, deg_ref, wnl_ref, wnr_ref, ws_ref, b_ref,
             o_ref):
    deg = deg_ref[0, :, :1] + deg_ref[1, :, :1]
    inv = 1.0 / jnp.maximum(deg, 1.0)
    h0 = acc_ref[0] * inv
    h1 = acc_ref[1] * inv
    x = feat_ref[...]
    dn = (((1,), (1,)), ((), ()))
    o_ref[...] = (
        lax.dot_general(x, ws_ref[...], dn, preferred_element_type=jnp.float32)
        + lax.dot_general(h0, wnl_ref[...], dn, preferred_element_type=jnp.float32)
        + lax.dot_general(h1, wnr_ref[...], dn, preferred_element_type=jnp.float32)
        + b_ref[...]
    )


ROW_BLK = 2000

_tc_combine = pl.pallas_call(
    _tc_body,
    grid=(N_NODES // ROW_BLK,),
    in_specs=[
        pl.BlockSpec((ROW_BLK, D), lambda i: (i, 0)),
        pl.BlockSpec((NC, ROW_BLK, DH), lambda i: (0, i, 0)),
        pl.BlockSpec((NC, ROW_BLK, DEG_W), lambda i: (0, i, 0)),
        pl.BlockSpec((D, DH), lambda i: (0, 0)),
        pl.BlockSpec((D, DH), lambda i: (0, 0)),
        pl.BlockSpec((D, D), lambda i: (0, 0)),
        pl.BlockSpec((1, D), lambda i: (0, 0)),
    ],
    out_specs=pl.BlockSpec((ROW_BLK, D), lambda i: (i, 0)),
    out_shape=jax.ShapeDtypeStruct((N_NODES, D), jnp.float32),
)


@jax.jit
def kernel(feat, edge_index, W_neigh, W_self, b_self):
    table = feat.reshape(N_NODES * 2, DH)
    zacc = jnp.zeros((ROWS_PER_TILE, DH), jnp.float32)
    zdeg = jnp.zeros((ROWS_PER_TILE, DEG_W), jnp.float32)
    ones = jnp.ones((CHUNK, DEG_W), jnp.float32)

    acc, deg = _sc_aggregate(edge_index, table, zacc, zdeg, ones)

    return _tc_combine(feat, acc, deg,
                       W_neigh[:, :DH], W_neigh[:, DH:],
                       W_self, b_self.reshape(1, D))
